# baseline (speedup 1.0000x reference)
import functools

import jax
import jax.numpy as jnp
from jax import lax
from jax.experimental import pallas as pl
from jax.experimental.pallas import tpu as pltpu
from jax.experimental.pallas import tpu_sc as plsc

_NC = 2
_NS = 16
_NW = _NC * _NS
_L = 16
_B = 1024
_D = 64
_P = 2 * _D
_BPW = _B // _NW


@functools.partial(
    pl.kernel,
    mesh=plsc.VectorSubcoreMesh(core_axis_name="c", subcore_axis_name="s"),
    out_type=jax.ShapeDtypeStruct((_NW, _L), jnp.float32),
    compiler_params=pltpu.CompilerParams(needs_layout_passes=False),
    scratch_types=[
        pltpu.VMEM((_BPW,), jnp.int32),
        pltpu.VMEM((_BPW,), jnp.int32),
        pltpu.VMEM((_BPW, _D), jnp.float32),
        pltpu.VMEM((_BPW, _P), jnp.float32),
        pltpu.VMEM((_L,), jnp.float32),
        pltpu.SemaphoreType.DMA,
    ],
)
def _center_mse_partials(features_hbm, labels_hbm, centers_hbm, out_hbm,
                         idx_v, gid_v, feat_v, rows_v, acc_v, sem):
    wid = lax.axis_index("s") * _NC + lax.axis_index("c")
    base = wid * _BPW
    pltpu.sync_copy(labels_hbm.at[pl.ds(base, _BPW)], idx_v)
    one = jnp.full((_L,), 1, jnp.int32)
    for c in range(_BPW // _L):
        gid_v[pl.ds(c * _L, _L)] = lax.shift_right_logical(
            idx_v[pl.ds(c * _L, _L)], one)
    gather = pltpu.async_copy(centers_hbm.at[gid_v], rows_v, sem)
    pltpu.sync_copy(features_hbm.at[pl.ds(base, _BPW)], feat_v)
    gather.wait()
    row_iota = lax.iota(jnp.int32, _L)
    acc = jnp.zeros((_L,), jnp.float32)
    for c in range(_BPW // _L):
        lbl = idx_v[pl.ds(c * _L, _L)]
        half = lax.shift_left(
            lax.bitwise_and(lbl, one), jnp.full((_L,), 6, jnp.int32))
        d0 = row_iota + c * _L
        for j in range(_D):
            col = half + j
            cv = plsc.load_gather(rows_v, [d0, col])
            fv = plsc.load_gather(feat_v, [d0, jnp.full((_L,), j, jnp.int32)])
            d = fv - cv
            acc = acc + d * d
    acc_v[...] = acc
    pltpu.sync_copy(acc_v, out_hbm.at[wid])


def kernel(features, labels, centers):
    centers2 = centers.reshape(centers.shape[0] // 2, _P)
    partials = _center_mse_partials(
        features, labels.astype(jnp.int32), centers2)
    return jnp.sum(partials) / jnp.float32(_B * _D)


# speedup vs baseline: 1.7095x; 1.7095x over previous
import functools

import jax
import jax.numpy as jnp
from jax import lax
from jax.experimental import pallas as pl
from jax.experimental.pallas import tpu as pltpu
from jax.experimental.pallas import tpu_sc as plsc

_NC = 2
_NS = 16
_NW = _NC * _NS
_L = 16
_B = 1024
_D = 64
_R = 8
_BPW = _B // _NW


@functools.partial(
    pl.kernel,
    mesh=plsc.VectorSubcoreMesh(core_axis_name="c", subcore_axis_name="s"),
    out_type=jax.ShapeDtypeStruct((_NW, _L), jnp.float32),
    compiler_params=pltpu.CompilerParams(needs_layout_passes=False),
    scratch_types=[
        pltpu.VMEM((_BPW,), jnp.int32),
        pltpu.VMEM((_BPW, _D), jnp.float32),
        pltpu.VMEM((_BPW, _R, _D), jnp.float32),
        pltpu.VMEM((_L,), jnp.float32),
        pltpu.SemaphoreType.DMA,
    ],
)
def _center_mse_partials(features_hbm, labels_hbm, centers_hbm, out_hbm,
                         idx_v, feat_v, rows_v, acc_v, sem):
    wid = lax.axis_index("s") * _NC + lax.axis_index("c")
    base = wid * _BPW
    pltpu.sync_copy(labels_hbm.at[pl.ds(base, _BPW)], idx_v)
    copies = []
    for c in range(_BPW // _L):
        gidv = lax.shift_right_logical(idx_v[pl.ds(c * _L, _L)], 3)
        for i in range(_L):
            copies.append(
                pltpu.async_copy(
                    centers_hbm.at[gidv[i]], rows_v.at[c * _L + i], sem))
    pltpu.sync_copy(features_hbm.at[pl.ds(base, _BPW)], feat_v)
    for cp in copies:
        cp.wait()
    row_iota = lax.iota(jnp.int32, _L)
    acc = jnp.zeros((_L,), jnp.float32)
    for c in range(_BPW // _L):
        lbl = idx_v[pl.ds(c * _L, _L)]
        sub = lax.bitwise_and(lbl, jnp.full((_L,), _R - 1, jnp.int32))
        d0 = row_iota + c * _L
        for j in range(_D):
            col = jnp.full((_L,), j, jnp.int32)
            cv = plsc.load_gather(rows_v, [d0, sub, col])
            fv = plsc.load_gather(feat_v, [d0, col])
            d = fv - cv
            acc = acc + d * d
    acc_v[...] = acc
    pltpu.sync_copy(acc_v, out_hbm.at[wid])


def kernel(features, labels, centers):
    centers3 = centers.reshape(centers.shape[0] // _R, _R, _D)
    partials = _center_mse_partials(
        features, labels.astype(jnp.int32), centers3)
    return jnp.sum(partials) / jnp.float32(_B * _D)
